# baseline (speedup 1.0000x reference)
import jax
import jax.numpy as jnp
from jax.experimental import pallas as pl

def _tiny_body(a_ref, o_ref):
    o_ref[...] = a_ref[...]

def kernel(x, author_embed, field_embed):
    t = pl.pallas_call(
        _tiny_body,
        in_specs=[pl.BlockSpec((8, 128), lambda: (0, 0))],
        out_specs=pl.BlockSpec((8, 128), lambda: (0, 0)),
        out_shape=jax.ShapeDtypeStruct((8, 128), jnp.float32),
    )(author_embed[:8])
    return (author_embed, field_embed, x, t)


# speedup vs baseline: 1.0664x; 1.0664x over previous
import jax
import jax.numpy as jnp
from jax.experimental import pallas as pl
from jax.experimental.pallas import tpu as pltpu

_GRID = 10
_ROWS_BIG = 100000 // _GRID
_ROWS_X = 50000 // _GRID
_D = 128


def _copy3_body(x_ref, a_ref, f_ref, ao_ref, fo_ref, xo_ref):
    ao_ref[...] = a_ref[...]
    fo_ref[...] = f_ref[...]
    xo_ref[...] = x_ref[...]


def kernel(x, author_embed, field_embed):
    out = pl.pallas_call(
        _copy3_body,
        grid=(_GRID,),
        in_specs=[
            pl.BlockSpec((_ROWS_X, _D), lambda i: (i, 0)),
            pl.BlockSpec((_ROWS_BIG, _D), lambda i: (i, 0)),
            pl.BlockSpec((_ROWS_BIG, _D), lambda i: (i, 0)),
        ],
        out_specs=[
            pl.BlockSpec((_ROWS_BIG, _D), lambda i: (i, 0)),
            pl.BlockSpec((_ROWS_BIG, _D), lambda i: (i, 0)),
            pl.BlockSpec((_ROWS_X, _D), lambda i: (i, 0)),
        ],
        out_shape=[
            jax.ShapeDtypeStruct(author_embed.shape, author_embed.dtype),
            jax.ShapeDtypeStruct(field_embed.shape, field_embed.dtype),
            jax.ShapeDtypeStruct(x.shape, x.dtype),
        ],
    )(x, author_embed, field_embed)
    return (out[0], out[1], out[2])
